# baseline (speedup 1.0000x reference)
import functools

import jax
import jax.numpy as jnp
from jax import lax
from jax.experimental import pallas as pl
from jax.experimental.pallas import tpu as pltpu
from jax.experimental.pallas import tpu_sc as plsc

_V = 1000000
_D = 32
_B = 16384 * 50
_NC, _NS = 2, 16
_NW = _NC * _NS
_ROWS_PER_W = _B // _NW
_GRP = 128
_NGRP = _ROWS_PER_W // _GRP
_CG = 4
_NCHUNK = _NGRP // _CG


def _make_gather():
  mesh = plsc.VectorSubcoreMesh(core_axis_name="c", subcore_axis_name="s")

  @functools.partial(
      pl.kernel,
      out_type=jax.ShapeDtypeStruct((4, _B // _GRP, 8, _GRP), jnp.float32),
      mesh=mesh,
      scratch_types=[
          pltpu.VMEM((_NGRP, _GRP), jnp.int32),
          pltpu.VMEM((_CG * _GRP, _D), jnp.float32),
          pltpu.VMEM((_CG * _GRP, _D), jnp.float32),
          pltpu.VMEM((4, _CG, 8, _GRP), jnp.float32),
          pltpu.VMEM((4, _CG, 8, _GRP), jnp.float32),
          pltpu.SemaphoreType.DMA,
          pltpu.SemaphoreType.DMA,
          pltpu.SemaphoreType.DMA,
          pltpu.SemaphoreType.DMA,
      ],
      compiler_params=pltpu.CompilerParams(
          use_tc_tiling_on_sc=False, needs_layout_passes=False),
  )
  def k(src_hbm, tab_hbm, out_hbm, idx_v, rows_a, rows_b, oblk_a, oblk_b,
        gsem_a, gsem_b, osem_a, osem_b):
    wid = lax.axis_index("s") * _NC + lax.axis_index("c")
    pltpu.sync_copy(src_hbm.at[wid], idx_v)
    gbase = wid * _NGRP
    iota16 = lax.iota(jnp.int32, 16)

    def fire_gathers(c, rows, sem):
      for g in range(_CG):
        pltpu.async_copy(
            tab_hbm.at[idx_v.at[c * _CG + g]],
            rows.at[pl.ds(g * _GRP, _GRP)], sem)

    def drain_g(sem, rows):
      pltpu.make_async_copy(tab_hbm.at[pl.ds(0, _CG * _GRP)], rows, sem).wait()

    def drain_o(sem, oblk):
      pltpu.make_async_copy(out_hbm.at[:, pl.ds(0, _CG)], oblk, sem).wait()

    def transpose_chunk(rows, oblk):
      @pl.loop(0, _CG)
      def _g(g):
        for m in range(8):
          row = iota16 + (g * _GRP + 16 * m)
          for r in range(4):
            for s in range(8):
              col = jnp.full((16,), 8 * r + s, jnp.int32)
              oblk[r, g, s, pl.ds(16 * m, 16)] = plsc.load_gather(
                  rows, [row, col])

    def fire_out(c, oblk, sem):
      pltpu.async_copy(
          oblk, out_hbm.at[:, pl.ds(gbase + c * _CG, _CG)], sem)

    fire_gathers(0, rows_a, gsem_a)

    @pl.loop(0, _NCHUNK, step=2)
    def _chunks(c0):
      drain_g(gsem_a, rows_a)
      fire_gathers(c0 + 1, rows_b, gsem_b)
      @pl.when(c0 >= 2)
      def _():
        drain_o(osem_a, oblk_a)
      transpose_chunk(rows_a, oblk_a)
      fire_out(c0, oblk_a, osem_a)
      drain_g(gsem_b, rows_b)
      @pl.when(c0 + 2 < _NCHUNK)
      def _():
        fire_gathers(c0 + 2, rows_a, gsem_a)
      @pl.when(c0 >= 1)
      def _():
        drain_o(osem_b, oblk_b)
      transpose_chunk(rows_b, oblk_b)
      fire_out(c0 + 1, oblk_b, osem_b)

    drain_o(osem_a, oblk_a)
    drain_o(osem_b, oblk_b)

  return k


_gather = _make_gather()


def kernel(inputs, embeddings):
  table_pk = jax.lax.optimization_barrier(
      jnp.reshape(embeddings, (_V // 4, 128)))
  tab = jnp.reshape(table_pk, (_V, _D))
  src = jnp.reshape(inputs.astype(jnp.int32), (_NW, _NGRP, _GRP))
  out4 = _gather(src, tab)
  return jnp.reshape(jnp.transpose(out4, (1, 3, 0, 2)), (_B, _D))
